# Initial kernel scaffold; baseline (speedup 1.0000x reference)
import jax
import jax.numpy as jnp
from jax.experimental import pallas as pl


def kernel(x, edge_index, edge_attr, batch, params):
    raise NotImplementedError("write your pallas kernel here")


# speedup vs baseline: 6.1686x; 6.1686x over previous
import functools

import jax
import jax.numpy as jnp
import numpy as np
from jax.experimental import pallas as pl

_N = 10000
_E = 320000
_G = 512
_HD = 128
_H = 8
_C = 16
_L = 4


def _att_mat(att):
    eye = jnp.eye(_H, dtype=jnp.float32)
    return (att[:, :, None] * eye[:, None, :]).reshape(_H * _C, _H)


def _linear_body(x_ref, w_ref, b_ref, o_ref, *, act):
    y = jnp.dot(x_ref[...], w_ref[...], preferred_element_type=jnp.float32)
    y = y + b_ref[...]
    if act == "relu":
        y = jnp.maximum(y, 0.0)
    o_ref[...] = y


def _linear(x, w, b, act=None, block=2048):
    m, k = x.shape
    n = w.shape[1]
    grid = -(-m // block)
    mp = grid * block
    xp = jnp.pad(x, ((0, mp - m), (0, 0))) if mp != m else x
    out = pl.pallas_call(
        functools.partial(_linear_body, act=act),
        grid=(grid,),
        in_specs=[
            pl.BlockSpec((block, k), lambda i: (i, 0)),
            pl.BlockSpec((k, n), lambda i: (0, 0)),
            pl.BlockSpec((1, n), lambda i: (0, 0)),
        ],
        out_specs=pl.BlockSpec((block, n), lambda i: (i, 0)),
        out_shape=jax.ShapeDtypeStruct((mp, n), jnp.float32),
    )(xp, w, b.reshape(1, n))
    return out[:m] if mp != m else out


def _proj_body(x_ref, w_ref, a_ref, xs_ref, a2_ref):
    xs = jnp.dot(x_ref[...], w_ref[...], preferred_element_type=jnp.float32)
    xs_ref[...] = xs
    a2_ref[...] = jnp.dot(xs, a_ref[...], preferred_element_type=jnp.float32)


def _proj(x, w, a, block=2048):
    m = x.shape[0]
    grid = -(-m // block)
    mp = grid * block
    xp = jnp.pad(x, ((0, mp - m), (0, 0))) if mp != m else x
    xs, a2 = pl.pallas_call(
        _proj_body,
        grid=(grid,),
        in_specs=[
            pl.BlockSpec((block, _HD), lambda i: (i, 0)),
            pl.BlockSpec((_HD, _HD), lambda i: (0, 0)),
            pl.BlockSpec((_HD, 2 * _H), lambda i: (0, 0)),
        ],
        out_specs=[
            pl.BlockSpec((block, _HD), lambda i: (i, 0)),
            pl.BlockSpec((block, 2 * _H), lambda i: (i, 0)),
        ],
        out_shape=[
            jax.ShapeDtypeStruct((mp, _HD), jnp.float32),
            jax.ShapeDtypeStruct((mp, 2 * _H), jnp.float32),
        ],
    )(xp, w, a)
    if mp != m:
        xs, a2 = xs[:m], a2[:m]
    return xs, a2


def _lrelu_body(s_ref, d_ref, e_ref, o_ref):
    a = s_ref[...] + d_ref[...] + e_ref[...]
    o_ref[...] = jnp.where(a >= 0.0, a, 0.2 * a)


_EM = _E * _H // 128


def _edge_lrelu(asg, adg, ae, block=2000):
    grid = _EM // block
    spec = pl.BlockSpec((block, 128), lambda i: (i, 0))
    out = pl.pallas_call(
        _lrelu_body,
        grid=(grid,),
        in_specs=[spec, spec, spec],
        out_specs=spec,
        out_shape=jax.ShapeDtypeStruct((_EM, 128), jnp.float32),
    )(asg.reshape(_EM, 128), adg.reshape(_EM, 128), ae.reshape(_EM, 128))
    return out.reshape(_E, _H)


def _exp_body(a_ref, m_ref, o_ref):
    o_ref[...] = jnp.exp(a_ref[...] - m_ref[...])


def _edge_exp(a, mg, block=2000):
    grid = _EM // block
    spec = pl.BlockSpec((block, 128), lambda i: (i, 0))
    out = pl.pallas_call(
        _exp_body,
        grid=(grid,),
        in_specs=[spec, spec],
        out_specs=spec,
        out_shape=jax.ShapeDtypeStruct((_EM, 128), jnp.float32),
    )(a.reshape(_EM, 128), mg.reshape(_EM, 128))
    return out.reshape(_E, _H)


def _epi_body(g_ref, s_ref, b_ref, o_ref):
    o_ref[...] = jnp.maximum(g_ref[...] * s_ref[...] + b_ref[...], 0.0)


def _epi_res_body(g_ref, s_ref, b_ref, r_ref, o_ref):
    y = jnp.maximum(g_ref[...] * s_ref[...] + b_ref[...], 0.0)
    o_ref[...] = y + r_ref[...]


def _epilogue(agg, es, eb, residual=None, block=2048):
    m = agg.shape[0]
    grid = -(-m // block)
    mp = grid * block
    pad = mp != m
    a = jnp.pad(agg, ((0, mp - m), (0, 0))) if pad else agg
    row = pl.BlockSpec((block, _HD), lambda i: (i, 0))
    vec = pl.BlockSpec((1, _HD), lambda i: (0, 0))
    if residual is None:
        out = pl.pallas_call(
            _epi_body,
            grid=(grid,),
            in_specs=[row, vec, vec],
            out_specs=row,
            out_shape=jax.ShapeDtypeStruct((mp, _HD), jnp.float32),
        )(a, es.reshape(1, _HD), eb.reshape(1, _HD))
    else:
        r = jnp.pad(residual, ((0, mp - m), (0, 0))) if pad else residual
        out = pl.pallas_call(
            _epi_res_body,
            grid=(grid,),
            in_specs=[row, vec, vec, row],
            out_specs=row,
            out_shape=jax.ShapeDtypeStruct((mp, _HD), jnp.float32),
        )(a, es.reshape(1, _HD), eb.reshape(1, _HD), r)
    return out[:m] if pad else out


def _gate_body(x_ref, w1_ref, b1_ref, w2_ref, b2_ref, o_ref):
    h = jnp.dot(x_ref[...], w1_ref[...], preferred_element_type=jnp.float32)
    h = jnp.maximum(h + b1_ref[...], 0.0)
    o_ref[...] = jnp.dot(h, w2_ref[...], preferred_element_type=jnp.float32) + b2_ref[...]


def _gate(x, w1, b1, w2, b2, block=2048):
    m = x.shape[0]
    grid = -(-m // block)
    mp = grid * block
    xp = jnp.pad(x, ((0, mp - m), (0, 0))) if mp != m else x
    hw = w1.shape[1]
    w2p = jnp.pad(w2, ((0, 0), (0, 8 - w2.shape[1])))
    b2p = jnp.pad(b2, ((0, 8 - b2.shape[0]),))
    out = pl.pallas_call(
        _gate_body,
        grid=(grid,),
        in_specs=[
            pl.BlockSpec((block, _HD), lambda i: (i, 0)),
            pl.BlockSpec((_HD, hw), lambda i: (0, 0)),
            pl.BlockSpec((1, hw), lambda i: (0, 0)),
            pl.BlockSpec((hw, 8), lambda i: (0, 0)),
            pl.BlockSpec((1, 8), lambda i: (0, 0)),
        ],
        out_specs=pl.BlockSpec((block, 8), lambda i: (i, 0)),
        out_shape=jax.ShapeDtypeStruct((mp, 8), jnp.float32),
    )(xp, w1, b1.reshape(1, hw), w2p, b2p.reshape(1, 8))
    return out[:m, 0]


def _head_body(x_ref, w1, b1, w2, b2, w3, b3, w4, b4, w5, b5, w6, b6, o_ref):
    def lin(v, w, b):
        return jnp.dot(v, w[...], preferred_element_type=jnp.float32) + b[...]

    h = jnp.maximum(lin(x_ref[...], w1, b1), 0.0)
    h = jnp.maximum(lin(h, w2, b2), 0.0)
    h = jnp.maximum(lin(h, w3, b3), 0.0)
    h = jnp.maximum(lin(h, w4, b4), 0.0)
    h = jnp.maximum(lin(h, w5, b5), 0.0)
    o_ref[...] = jax.nn.sigmoid(lin(h, w6, b6))


def _head(gr, layers):
    ws = []
    specs = [pl.BlockSpec(gr.shape, lambda: (0, 0))]
    for w, b in layers:
        ws.append(w)
        ws.append(b.reshape(1, -1))
        specs.append(pl.BlockSpec(w.shape, lambda: (0, 0)))
        specs.append(pl.BlockSpec((1, w.shape[1]), lambda: (0, 0)))
    out = pl.pallas_call(
        _head_body,
        in_specs=specs,
        out_specs=pl.BlockSpec((_G, 8), lambda: (0, 0)),
        out_shape=jax.ShapeDtypeStruct((_G, 8), jnp.float32),
    )(gr, *ws)
    return out[:, :1]


def kernel(x, edge_index, edge_attr, batch, params):
    p = params
    src = edge_index[0]
    dst = edge_index[1]

    xh = _linear(x, p["node_W"], p["node_b"])

    eWs, ebs = [], []
    for g in p["gats"]:
        me = g["We"] @ _att_mat(g["att_edge"])
        eWs.append(p["edge_W"] @ me)
        ebs.append(p["edge_b"] @ me)
    a_edge_all = _linear(
        edge_attr, jnp.concatenate(eWs, axis=1), jnp.concatenate(ebs, axis=0),
        block=4000,
    )

    inv = 1.0 / np.sqrt(1.0 + 1e-5)
    residual = xh
    xc = xh
    for i in range(_L):
        g = p["gats"][i]
        amat = jnp.concatenate(
            [_att_mat(g["att_src"]), _att_mat(g["att_dst"])], axis=1)
        xs, a2 = _proj(xc, g["W"], amat)
        a_src, a_dst = a2[:, :_H], a2[:, _H:]

        a = _edge_lrelu(a_src[src], a_dst[dst], a_edge_all[:, i * _H:(i + 1) * _H])
        m = jax.ops.segment_max(a, dst, num_segments=_N)
        ex = _edge_exp(a, m[dst])
        denom = jax.ops.segment_sum(ex, dst, num_segments=_N)
        alpha = ex / (denom[dst] + 1e-16)
        msg = (xs[src].reshape(_E, _H, _C) * alpha[:, :, None]).reshape(_E, _HD)
        agg = jax.ops.segment_sum(msg, dst, num_segments=_N)

        bn = p["bns"][i]
        es = bn["gamma"] * inv
        eb = g["b"] * es + bn["beta"]
        if i > 0 and i % 2 == 0:
            xc = _epilogue(agg, es, eb, residual=residual)
            residual = xc
        else:
            xc = _epilogue(agg, es, eb)

    ones = jnp.ones((_N,), jnp.float32)
    counts = jax.ops.segment_sum(ones, batch, num_segments=_G)
    x_mean = jax.ops.segment_sum(xc, batch, num_segments=_G) / jnp.maximum(
        counts, 1.0)[:, None]
    x_max = jax.ops.segment_max(xc, batch, num_segments=_G)
    x_max = jnp.where(counts[:, None] > 0, x_max, 0.0)

    gate = _gate(xc, p["gate1_W"], p["gate1_b"], p["gate2_W"], p["gate2_b"])
    gm = jax.ops.segment_max(gate, batch, num_segments=_G)
    ge = jnp.exp(gate - gm[batch])
    gs = jax.ops.segment_sum(ge, batch, num_segments=_G)
    w = ge / (gs[batch] + 1e-16)
    x_att = jax.ops.segment_sum(w[:, None] * xc, batch, num_segments=_G)

    gr = jnp.concatenate([x_mean, x_max, x_att], axis=1)
    cl2_wp = jnp.pad(p["cl2_W"], ((0, 0), (0, 7)))
    cl2_bp = jnp.pad(p["cl2_b"], ((0, 7),))
    return _head(gr, [
        (p["mlp1_W"], p["mlp1_b"]),
        (p["mlp2_W"], p["mlp2_b"]),
        (p["fe1_W"], p["fe1_b"]),
        (p["fe2_W"], p["fe2_b"]),
        (p["cl1_W"], p["cl1_b"]),
        (cl2_wp, cl2_bp),
    ])


# speedup vs baseline: 7.0099x; 1.1364x over previous
import functools

import jax
import jax.numpy as jnp
import numpy as np
from jax.experimental import pallas as pl

_N = 10000
_E = 320000
_G = 512
_HD = 128
_H = 8
_C = 16
_L = 4


def _att_mat(att):
    eye = jnp.eye(_H, dtype=jnp.float32)
    return (att[:, :, None] * eye[:, None, :]).reshape(_H * _C, _H)


def _linear_body(x_ref, w_ref, b_ref, o_ref, *, act):
    y = jnp.dot(x_ref[...], w_ref[...], preferred_element_type=jnp.float32)
    y = y + b_ref[...]
    if act == "relu":
        y = jnp.maximum(y, 0.0)
    o_ref[...] = y


def _linear(x, w, b, act=None, block=2048):
    m, k = x.shape
    n = w.shape[1]
    grid = -(-m // block)
    mp = grid * block
    xp = jnp.pad(x, ((0, mp - m), (0, 0))) if mp != m else x
    out = pl.pallas_call(
        functools.partial(_linear_body, act=act),
        grid=(grid,),
        in_specs=[
            pl.BlockSpec((block, k), lambda i: (i, 0)),
            pl.BlockSpec((k, n), lambda i: (0, 0)),
            pl.BlockSpec((1, n), lambda i: (0, 0)),
        ],
        out_specs=pl.BlockSpec((block, n), lambda i: (i, 0)),
        out_shape=jax.ShapeDtypeStruct((mp, n), jnp.float32),
    )(xp, w, b.reshape(1, n))
    return out[:m] if mp != m else out


def _proj_body(x_ref, w_ref, a_ref, xs_ref, a2_ref):
    xs = jnp.dot(x_ref[...], w_ref[...], preferred_element_type=jnp.float32)
    xs_ref[...] = xs
    a2_ref[...] = jnp.dot(xs, a_ref[...], preferred_element_type=jnp.float32)


def _proj(x, w, a, block=2048):
    m = x.shape[0]
    grid = -(-m // block)
    mp = grid * block
    xp = jnp.pad(x, ((0, mp - m), (0, 0))) if mp != m else x
    xs, a2 = pl.pallas_call(
        _proj_body,
        grid=(grid,),
        in_specs=[
            pl.BlockSpec((block, _HD), lambda i: (i, 0)),
            pl.BlockSpec((_HD, _HD), lambda i: (0, 0)),
            pl.BlockSpec((_HD, 2 * _H), lambda i: (0, 0)),
        ],
        out_specs=[
            pl.BlockSpec((block, _HD), lambda i: (i, 0)),
            pl.BlockSpec((block, 2 * _H), lambda i: (i, 0)),
        ],
        out_shape=[
            jax.ShapeDtypeStruct((mp, _HD), jnp.float32),
            jax.ShapeDtypeStruct((mp, 2 * _H), jnp.float32),
        ],
    )(xp, w, a)
    if mp != m:
        xs, a2 = xs[:m], a2[:m]
    return xs, a2


def _lrelu_body(s_ref, d_ref, e_ref, o_ref):
    a = s_ref[...] + d_ref[...] + e_ref[...]
    o_ref[...] = jnp.where(a >= 0.0, a, 0.2 * a)


_EM = _E * _H // 128


def _edge_lrelu(asg, adg, ae, block=2000):
    grid = _EM // block
    spec = pl.BlockSpec((block, 128), lambda i: (i, 0))
    out = pl.pallas_call(
        _lrelu_body,
        grid=(grid,),
        in_specs=[spec, spec, spec],
        out_specs=spec,
        out_shape=jax.ShapeDtypeStruct((_EM, 128), jnp.float32),
    )(asg.reshape(_EM, 128), adg.reshape(_EM, 128), ae.reshape(_EM, 128))
    return out.reshape(_E, _H)


def _exp_body(a_ref, m_ref, o_ref):
    o_ref[...] = jnp.exp(a_ref[...] - m_ref[...])


def _edge_exp(a, mg, block=2000):
    grid = _EM // block
    spec = pl.BlockSpec((block, 128), lambda i: (i, 0))
    out = pl.pallas_call(
        _exp_body,
        grid=(grid,),
        in_specs=[spec, spec],
        out_specs=spec,
        out_shape=jax.ShapeDtypeStruct((_EM, 128), jnp.float32),
    )(a.reshape(_EM, 128), mg.reshape(_EM, 128))
    return out.reshape(_E, _H)


def _msg_body(xs_ref, ex_ref, dn_ref, exp_ref, o_ref):
    alpha = ex_ref[...] / (dn_ref[...] + 1e-16)
    o_ref[...] = xs_ref[...] * jnp.dot(
        alpha, exp_ref[...], preferred_element_type=jnp.float32)


def _msg(xs_g, ex, dn, block=4000):
    grid = _E // block
    expand = jnp.repeat(jnp.eye(_H, dtype=jnp.float32), _C, axis=1)
    row = pl.BlockSpec((block, _HD), lambda i: (i, 0))
    nar = pl.BlockSpec((block, _H), lambda i: (i, 0))
    return pl.pallas_call(
        _msg_body,
        grid=(grid,),
        in_specs=[row, nar, nar, pl.BlockSpec((_H, _HD), lambda i: (0, 0))],
        out_specs=row,
        out_shape=jax.ShapeDtypeStruct((_E, _HD), jnp.float32),
    )(xs_g, ex, dn, expand)


def _epi_body(g_ref, s_ref, b_ref, o_ref):
    o_ref[...] = jnp.maximum(g_ref[...] * s_ref[...] + b_ref[...], 0.0)


def _epi_res_body(g_ref, s_ref, b_ref, r_ref, o_ref):
    y = jnp.maximum(g_ref[...] * s_ref[...] + b_ref[...], 0.0)
    o_ref[...] = y + r_ref[...]


def _epilogue(agg, es, eb, residual=None, block=2048):
    m = agg.shape[0]
    grid = -(-m // block)
    mp = grid * block
    pad = mp != m
    a = jnp.pad(agg, ((0, mp - m), (0, 0))) if pad else agg
    row = pl.BlockSpec((block, _HD), lambda i: (i, 0))
    vec = pl.BlockSpec((1, _HD), lambda i: (0, 0))
    if residual is None:
        out = pl.pallas_call(
            _epi_body,
            grid=(grid,),
            in_specs=[row, vec, vec],
            out_specs=row,
            out_shape=jax.ShapeDtypeStruct((mp, _HD), jnp.float32),
        )(a, es.reshape(1, _HD), eb.reshape(1, _HD))
    else:
        r = jnp.pad(residual, ((0, mp - m), (0, 0))) if pad else residual
        out = pl.pallas_call(
            _epi_res_body,
            grid=(grid,),
            in_specs=[row, vec, vec, row],
            out_specs=row,
            out_shape=jax.ShapeDtypeStruct((mp, _HD), jnp.float32),
        )(a, es.reshape(1, _HD), eb.reshape(1, _HD), r)
    return out[:m] if pad else out


def _gate_body(x_ref, w1_ref, b1_ref, w2_ref, b2_ref, o_ref):
    h = jnp.dot(x_ref[...], w1_ref[...], preferred_element_type=jnp.float32)
    h = jnp.maximum(h + b1_ref[...], 0.0)
    o_ref[...] = jnp.dot(h, w2_ref[...], preferred_element_type=jnp.float32) + b2_ref[...]


def _gate(x, w1, b1, w2, b2, block=2048):
    m = x.shape[0]
    grid = -(-m // block)
    mp = grid * block
    xp = jnp.pad(x, ((0, mp - m), (0, 0))) if mp != m else x
    hw = w1.shape[1]
    w2p = jnp.pad(w2, ((0, 0), (0, 8 - w2.shape[1])))
    b2p = jnp.pad(b2, ((0, 8 - b2.shape[0]),))
    out = pl.pallas_call(
        _gate_body,
        grid=(grid,),
        in_specs=[
            pl.BlockSpec((block, _HD), lambda i: (i, 0)),
            pl.BlockSpec((_HD, hw), lambda i: (0, 0)),
            pl.BlockSpec((1, hw), lambda i: (0, 0)),
            pl.BlockSpec((hw, 8), lambda i: (0, 0)),
            pl.BlockSpec((1, 8), lambda i: (0, 0)),
        ],
        out_specs=pl.BlockSpec((block, 8), lambda i: (i, 0)),
        out_shape=jax.ShapeDtypeStruct((mp, 8), jnp.float32),
    )(xp, w1, b1.reshape(1, hw), w2p, b2p.reshape(1, 8))
    return out[:m, 0]


def _head_body(x_ref, w1, b1, w2, b2, w3, b3, w4, b4, w5, b5, w6, b6, o_ref):
    def lin(v, w, b):
        return jnp.dot(v, w[...], preferred_element_type=jnp.float32) + b[...]

    h = jnp.maximum(lin(x_ref[...], w1, b1), 0.0)
    h = jnp.maximum(lin(h, w2, b2), 0.0)
    h = jnp.maximum(lin(h, w3, b3), 0.0)
    h = jnp.maximum(lin(h, w4, b4), 0.0)
    h = jnp.maximum(lin(h, w5, b5), 0.0)
    o_ref[...] = jax.nn.sigmoid(lin(h, w6, b6))


def _head(gr, layers):
    ws = []
    specs = [pl.BlockSpec(gr.shape, lambda: (0, 0))]
    for w, b in layers:
        ws.append(w)
        ws.append(b.reshape(1, -1))
        specs.append(pl.BlockSpec(w.shape, lambda: (0, 0)))
        specs.append(pl.BlockSpec((1, w.shape[1]), lambda: (0, 0)))
    out = pl.pallas_call(
        _head_body,
        in_specs=specs,
        out_specs=pl.BlockSpec((_G, 8), lambda: (0, 0)),
        out_shape=jax.ShapeDtypeStruct((_G, 8), jnp.float32),
    )(gr, *ws)
    return out[:, :1]


def kernel(x, edge_index, edge_attr, batch, params):
    p = params
    src = edge_index[0]
    dst = edge_index[1]

    xh = _linear(x, p["node_W"], p["node_b"])

    eWs, ebs = [], []
    for g in p["gats"]:
        me = g["We"] @ _att_mat(g["att_edge"])
        eWs.append(p["edge_W"] @ me)
        ebs.append(p["edge_b"] @ me)
    a_edge_all = _linear(
        edge_attr, jnp.concatenate(eWs, axis=1), jnp.concatenate(ebs, axis=0),
        block=4000,
    )

    inv = 1.0 / np.sqrt(1.0 + 1e-5)
    residual = xh
    xc = xh
    for i in range(_L):
        g = p["gats"][i]
        amat = jnp.concatenate(
            [_att_mat(g["att_src"]), _att_mat(g["att_dst"])], axis=1)
        xs, a2 = _proj(xc, g["W"], amat)
        a_src, a_dst = a2[:, :_H], a2[:, _H:]

        a = _edge_lrelu(a_src[src], a_dst[dst], a_edge_all[:, i * _H:(i + 1) * _H])
        m = jax.ops.segment_max(a, dst, num_segments=_N)
        ex = _edge_exp(a, m[dst])
        denom = jax.ops.segment_sum(ex, dst, num_segments=_N)
        msg = _msg(xs[src], ex, denom[dst])
        agg = jax.ops.segment_sum(msg, dst, num_segments=_N)

        bn = p["bns"][i]
        es = bn["gamma"] * inv
        eb = g["b"] * es + bn["beta"]
        if i > 0 and i % 2 == 0:
            xc = _epilogue(agg, es, eb, residual=residual)
            residual = xc
        else:
            xc = _epilogue(agg, es, eb)

    ones = jnp.ones((_N,), jnp.float32)
    counts = jax.ops.segment_sum(ones, batch, num_segments=_G)
    x_mean = jax.ops.segment_sum(xc, batch, num_segments=_G) / jnp.maximum(
        counts, 1.0)[:, None]
    x_max = jax.ops.segment_max(xc, batch, num_segments=_G)
    x_max = jnp.where(counts[:, None] > 0, x_max, 0.0)

    gate = _gate(xc, p["gate1_W"], p["gate1_b"], p["gate2_W"], p["gate2_b"])
    gm = jax.ops.segment_max(gate, batch, num_segments=_G)
    ge = jnp.exp(gate - gm[batch])
    gs = jax.ops.segment_sum(ge, batch, num_segments=_G)
    w = ge / (gs[batch] + 1e-16)
    x_att = jax.ops.segment_sum(w[:, None] * xc, batch, num_segments=_G)

    gr = jnp.concatenate([x_mean, x_max, x_att], axis=1)
    cl2_wp = jnp.pad(p["cl2_W"], ((0, 0), (0, 7)))
    cl2_bp = jnp.pad(p["cl2_b"], ((0, 7),))
    return _head(gr, [
        (p["mlp1_W"], p["mlp1_b"]),
        (p["mlp2_W"], p["mlp2_b"]),
        (p["fe1_W"], p["fe1_b"]),
        (p["fe2_W"], p["fe2_b"]),
        (p["cl1_W"], p["cl1_b"]),
        (cl2_wp, cl2_bp),
    ])
